# baseline (speedup 1.0000x reference)
import functools

import jax
import jax.numpy as jnp
from jax import lax
from jax.experimental import pallas as pl
from jax.experimental.pallas import tpu as pltpu
from jax.experimental.pallas import tpu_sc as plsc

R = 16
NBUF = 5
UNROLL = 8


def kernel(x, pos_emb):
    B, T, C = x.shape
    info = plsc.get_sparse_core_info()
    NW = info.num_cores * info.num_subcores
    tw = T // NW
    nchunks = tw // R
    nitems = nchunks * B
    CPG = C // 16

    mesh = plsc.VectorSubcoreMesh(core_axis_name="c", subcore_axis_name="s")

    @functools.partial(
        pl.kernel,
        mesh=mesh,
        out_type=jax.ShapeDtypeStruct((B, T, C), jnp.float32),
        scratch_types=[
            pltpu.VMEM((NBUF, R, C), jnp.float32),
            pltpu.VMEM((2, R, C), jnp.float32),
            pltpu.SemaphoreType.DMA,
            pltpu.SemaphoreType.DMA,
            pltpu.SemaphoreType.DMA,
        ],
    )
    def k(x_hbm, pe_hbm, out_hbm, xb, peb, xsem, pesem, osem):
        wid = lax.axis_index("s") * info.num_cores + lax.axis_index("c")
        t0 = wid * tw

        def trow(item):
            return t0 + (item // B) * R

        pltpu.async_copy(pe_hbm.at[pl.ds(t0, R)], peb.at[0], pesem)
        pltpu.async_copy(pe_hbm.at[pl.ds(t0 + R, R)], peb.at[1], pesem)
        pltpu.async_copy(x_hbm.at[0, pl.ds(t0, R)], xb.at[0], xsem)
        pltpu.async_copy(x_hbm.at[1, pl.ds(t0, R)], xb.at[1], xsem)

        def body(kk, _):
            i = kk // B
            b = kk % B
            slot = kk % NBUF
            pslot = i % 2

            @pl.when(kk >= 3)
            def _():
                pltpu.make_async_copy(
                    xb.at[(kk - 3) % NBUF],
                    out_hbm.at[(kk - 3) % B, pl.ds(trow(kk - 3), R)],
                    osem,
                ).wait()

            @pl.when(kk + 2 < nitems)
            def _():
                pltpu.async_copy(
                    x_hbm.at[(kk + 2) % B, pl.ds(trow(kk + 2), R)],
                    xb.at[(kk + 2) % NBUF],
                    xsem,
                )

            pltpu.make_async_copy(
                x_hbm.at[b, pl.ds(trow(kk), R)], xb.at[slot], xsem
            ).wait()

            @pl.when(b == 0)
            def _():
                pltpu.make_async_copy(
                    pe_hbm.at[pl.ds(trow(kk), R)], peb.at[pslot], pesem
                ).wait()

            @plsc.parallel_loop(0, R * CPG, unroll=UNROLL)
            def _(g):
                row = g // CPG
                col = (g % CPG) * 16
                plsc.addupdate(
                    xb.at[slot, row, pl.ds(col, 16)],
                    peb[pslot, row, pl.ds(col, 16)],
                )

            @pl.when((b == B - 1) & (i + 2 < nchunks))
            def _():
                pltpu.async_copy(
                    pe_hbm.at[pl.ds(t0 + (i + 2) * R, R)], peb.at[pslot], pesem
                )

            pltpu.async_copy(
                xb.at[slot], out_hbm.at[b, pl.ds(trow(kk), R)], osem
            )
            return 0

        lax.fori_loop(0, nitems, body, 0)
        for d in (3, 2, 1):
            pltpu.make_async_copy(
                xb.at[(nitems - d) % NBUF],
                out_hbm.at[(nitems - d) % B, pl.ds(trow(nitems - d), R)],
                osem,
            ).wait()

    return k(x, pos_emb[:T])


# speedup vs baseline: 1.0015x; 1.0015x over previous
import functools

import jax
import jax.numpy as jnp
from jax import lax
from jax.experimental import pallas as pl
from jax.experimental.pallas import tpu as pltpu
from jax.experimental.pallas import tpu_sc as plsc

R = 8
NBUF = 8
UNROLL = 8


def kernel(x, pos_emb):
    B, T, C = x.shape
    info = plsc.get_sparse_core_info()
    NW = info.num_cores * info.num_subcores
    tw = T // NW
    nchunks = tw // R
    nitems = nchunks * B
    CPG = C // 16

    mesh = plsc.VectorSubcoreMesh(core_axis_name="c", subcore_axis_name="s")

    @functools.partial(
        pl.kernel,
        mesh=mesh,
        out_type=jax.ShapeDtypeStruct((B, T, C), jnp.float32),
        scratch_types=[
            pltpu.VMEM((NBUF, R, C), jnp.float32),
            pltpu.VMEM((2, R, C), jnp.float32),
            pltpu.SemaphoreType.DMA,
            pltpu.SemaphoreType.DMA,
            pltpu.SemaphoreType.DMA,
        ],
    )
    def k(x_hbm, pe_hbm, out_hbm, xb, peb, xsem, pesem, osem):
        wid = lax.axis_index("s") * info.num_cores + lax.axis_index("c")
        t0 = wid * tw

        def trow(item):
            return t0 + (item // B) * R

        pltpu.async_copy(pe_hbm.at[pl.ds(t0, R)], peb.at[0], pesem)
        pltpu.async_copy(pe_hbm.at[pl.ds(t0 + R, R)], peb.at[1], pesem)
        pltpu.async_copy(x_hbm.at[0, pl.ds(t0, R)], xb.at[0], xsem)
        pltpu.async_copy(x_hbm.at[1, pl.ds(t0, R)], xb.at[1], xsem)
        pltpu.async_copy(x_hbm.at[2, pl.ds(t0, R)], xb.at[2], xsem)

        def body(kk, _):
            i = kk // B
            b = kk % B
            slot = kk % NBUF
            pslot = i % 2

            @pl.when(kk >= 5)
            def _():
                pltpu.make_async_copy(
                    xb.at[(kk - 5) % NBUF],
                    out_hbm.at[(kk - 5) % B, pl.ds(trow(kk - 5), R)],
                    osem,
                ).wait()

            @pl.when(kk + 3 < nitems)
            def _():
                pltpu.async_copy(
                    x_hbm.at[(kk + 3) % B, pl.ds(trow(kk + 3), R)],
                    xb.at[(kk + 3) % NBUF],
                    xsem,
                )

            pltpu.make_async_copy(
                x_hbm.at[b, pl.ds(trow(kk), R)], xb.at[slot], xsem
            ).wait()

            @pl.when(b == 0)
            def _():
                pltpu.make_async_copy(
                    pe_hbm.at[pl.ds(trow(kk), R)], peb.at[pslot], pesem
                ).wait()

            @plsc.parallel_loop(0, R * CPG, unroll=UNROLL)
            def _(g):
                row = g // CPG
                col = (g % CPG) * 16
                plsc.addupdate(
                    xb.at[slot, row, pl.ds(col, 16)],
                    peb[pslot, row, pl.ds(col, 16)],
                )

            @pl.when((b == B - 1) & (i + 2 < nchunks))
            def _():
                pltpu.async_copy(
                    pe_hbm.at[pl.ds(t0 + (i + 2) * R, R)], peb.at[pslot], pesem
                )

            pltpu.async_copy(
                xb.at[slot], out_hbm.at[b, pl.ds(trow(kk), R)], osem
            )
            return 0

        lax.fori_loop(0, nitems, body, 0)
        for d in (5, 4, 3, 2, 1):
            pltpu.make_async_copy(
                xb.at[(nitems - d) % NBUF],
                out_hbm.at[(nitems - d) % B, pl.ds(trow(nitems - d), R)],
                osem,
            ).wait()

    return k(x, pos_emb[:T])
